# baseline (device time: 58276 ns/iter reference)
import jax
import jax.numpy as jnp
from jax import lax
from jax.experimental import pallas as pl
from jax.experimental.pallas import tpu as pltpu

N_DEV = 16
K_SUB = 8


def kernel(x):
    _, m, n_total = x.shape
    n_chunk = n_total // N_DEV
    m_half = m // 2
    m_sub = m_half // K_SUB

    def body(
        x_ref, out_ref,
        cw_ref, ccw_ref,
        cw_send, cw_recv, ccw_send, ccw_recv,
    ):
        my = lax.axis_index("i")

        q = lax.rem(my, 4)
        z = my // 4
        r = jnp.where(
            q == 0, z,
            jnp.where(q == 1, 7 - z, jnp.where(q == 2, 8 + z, 15 - z)),
        )

        def dev_at(rho):
            rho = lax.rem(rho + 2 * N_DEV, N_DEV)
            col = rho // 4
            off = lax.rem(rho, 4)
            return jnp.where(
                col == 0, 4 * off,
                jnp.where(
                    col == 1, 4 * (3 - off) + 1,
                    jnp.where(col == 2, 4 * off + 2, 4 * (3 - off) + 3),
                ),
            )

        left = dev_at(r - 1)
        right = dev_at(r + 1)

        barrier_sem = pltpu.get_barrier_semaphore()
        for nbr in (left, right):
            pl.semaphore_signal(
                barrier_sem, inc=1,
                device_id=(nbr,), device_id_type=pl.DeviceIdType.MESH,
            )
        pl.semaphore_wait(barrier_sem, 2)

        def cw_x(c, j):
            return x_ref[
                0, j * m_sub:(j + 1) * m_sub, pl.ds(c * n_chunk, n_chunk)
            ].astype(jnp.bfloat16)

        def ccw_x(c, j):
            return x_ref[
                0,
                m_half + j * m_sub:m_half + (j + 1) * m_sub,
                pl.ds(c * n_chunk, n_chunk),
            ].astype(jnp.bfloat16)

        def rows(buf_slot, j):
            return buf_slot.at[pl.ds(j * m_sub, m_sub), :]

        def make(dirn, s, j):
            buf, send, recv, tgt = {
                "cw": (cw_ref, cw_send, cw_recv, right),
                "ccw": (ccw_ref, ccw_send, ccw_recv, left),
            }[dirn]
            src_slot = (N_DEV - 1) if s == 0 else (s - 1)
            return pltpu.make_async_remote_copy(
                src_ref=rows(buf.at[src_slot], j),
                dst_ref=rows(buf.at[s], j),
                send_sem=send.at[s, j],
                recv_sem=recv.at[s, j],
                device_id=(tgt,),
                device_id_type=pl.DeviceIdType.MESH,
            )

        cw_ref[N_DEV - 1, :, :] = (
            x_ref[0, 0:m_half, pl.ds(dev_at(r - 1) * n_chunk, n_chunk)]
        ).astype(jnp.bfloat16)
        ccw_ref[N_DEV - 1, :, :] = (
            x_ref[0, m_half:m, pl.ds(dev_at(r + 1) * n_chunk, n_chunk)]
        ).astype(jnp.bfloat16)

        rdmas = {}
        for j in range(K_SUB):
            for dirn in ("cw", "ccw"):
                rdmas[(dirn, 0, j)] = make(dirn, 0, j)
                rdmas[(dirn, 0, j)].start()

        for s in range(1, N_DEV - 1):
            c_cw = dev_at(r - 1 - s)
            c_ccw = dev_at(r + 1 + s)
            for j in range(K_SUB):
                lo = j * m_sub
                hi = (j + 1) * m_sub
                xj_cw = cw_x(c_cw, j)
                xj_ccw = ccw_x(c_ccw, j)
                rdmas[("cw", s - 1, j)].wait_recv()
                cw_ref[s - 1, lo:hi, :] = cw_ref[s - 1, lo:hi, :] + xj_cw
                rdmas[("cw", s, j)] = make("cw", s, j)
                rdmas[("cw", s, j)].start()
                rdmas[("ccw", s - 1, j)].wait_recv()
                ccw_ref[s - 1, lo:hi, :] = ccw_ref[s - 1, lo:hi, :] + xj_ccw
                rdmas[("ccw", s, j)] = make("ccw", s, j)
                rdmas[("ccw", s, j)].start()

        for j in range(K_SUB):
            lo = j * m_sub
            hi = (j + 1) * m_sub
            rdmas[("cw", N_DEV - 2, j)].wait_recv()
            out_ref[lo:hi, :] = (
                cw_ref[N_DEV - 2, lo:hi, :].astype(jnp.float32)
                + x_ref[0, lo:hi, pl.ds(my * n_chunk, n_chunk)]
            )
            rdmas[("ccw", N_DEV - 2, j)].wait_recv()
            out_ref[m_half + lo:m_half + hi, :] = (
                ccw_ref[N_DEV - 2, lo:hi, :].astype(jnp.float32)
                + x_ref[0, m_half + lo:m_half + hi, pl.ds(my * n_chunk, n_chunk)]
            )

        for key in rdmas:
            rdmas[key].wait_send()

    return pl.pallas_call(
        body,
        out_shape=jax.ShapeDtypeStruct((m, n_chunk), jnp.float32),
        in_specs=[pl.BlockSpec(memory_space=pltpu.VMEM)],
        out_specs=pl.BlockSpec(memory_space=pltpu.VMEM),
        scratch_shapes=[
            pltpu.VMEM((N_DEV, m_half, n_chunk), jnp.bfloat16),
            pltpu.VMEM((N_DEV, m_half, n_chunk), jnp.bfloat16),
            pltpu.SemaphoreType.DMA((N_DEV - 1, K_SUB)),
            pltpu.SemaphoreType.DMA((N_DEV - 1, K_SUB)),
            pltpu.SemaphoreType.DMA((N_DEV - 1, K_SUB)),
            pltpu.SemaphoreType.DMA((N_DEV - 1, K_SUB)),
        ],
        compiler_params=pltpu.CompilerParams(collective_id=0),
    )(x)


# device time: 56306 ns/iter; 1.0350x vs baseline; 1.0350x over previous
import jax
import jax.numpy as jnp
from jax import lax
from jax.experimental import pallas as pl
from jax.experimental.pallas import tpu as pltpu

N_DEV = 16
K_SUB = 4


def kernel(x):
    _, m, n_total = x.shape
    n_chunk = n_total // N_DEV
    m_half = m // 2
    m_sub = m_half // K_SUB

    def body(
        x_ref, out_ref,
        cw_ref, ccw_ref,
        cw_send, cw_recv, ccw_send, ccw_recv,
    ):
        my = lax.axis_index("i")

        q = lax.rem(my, 4)
        z = my // 4
        r = jnp.where(
            q == 0, z,
            jnp.where(q == 1, 7 - z, jnp.where(q == 2, 8 + z, 15 - z)),
        )

        def dev_at(rho):
            rho = lax.rem(rho + 2 * N_DEV, N_DEV)
            col = rho // 4
            off = lax.rem(rho, 4)
            return jnp.where(
                col == 0, 4 * off,
                jnp.where(
                    col == 1, 4 * (3 - off) + 1,
                    jnp.where(col == 2, 4 * off + 2, 4 * (3 - off) + 3),
                ),
            )

        left = dev_at(r - 1)
        right = dev_at(r + 1)

        barrier_sem = pltpu.get_barrier_semaphore()
        for nbr in (left, right):
            pl.semaphore_signal(
                barrier_sem, inc=1,
                device_id=(nbr,), device_id_type=pl.DeviceIdType.MESH,
            )
        pl.semaphore_wait(barrier_sem, 2)

        def cw_x(c, j):
            return x_ref[
                0, j * m_sub:(j + 1) * m_sub, pl.ds(c * n_chunk, n_chunk)
            ].astype(jnp.bfloat16)

        def ccw_x(c, j):
            return x_ref[
                0,
                m_half + j * m_sub:m_half + (j + 1) * m_sub,
                pl.ds(c * n_chunk, n_chunk),
            ].astype(jnp.bfloat16)

        def rows(buf_slot, j):
            return buf_slot.at[pl.ds(j * m_sub, m_sub), :]

        def make(dirn, s, j):
            buf, send, recv, tgt = {
                "cw": (cw_ref, cw_send, cw_recv, right),
                "ccw": (ccw_ref, ccw_send, ccw_recv, left),
            }[dirn]
            src_slot = (N_DEV - 1) if s == 0 else (s - 1)
            return pltpu.make_async_remote_copy(
                src_ref=rows(buf.at[src_slot], j),
                dst_ref=rows(buf.at[s], j),
                send_sem=send.at[s, j],
                recv_sem=recv.at[s, j],
                device_id=(tgt,),
                device_id_type=pl.DeviceIdType.MESH,
            )

        cw_ref[N_DEV - 1, :, :] = (
            x_ref[0, 0:m_half, pl.ds(dev_at(r - 1) * n_chunk, n_chunk)]
        ).astype(jnp.bfloat16)
        ccw_ref[N_DEV - 1, :, :] = (
            x_ref[0, m_half:m, pl.ds(dev_at(r + 1) * n_chunk, n_chunk)]
        ).astype(jnp.bfloat16)

        rdmas = {}
        for j in range(K_SUB):
            for dirn in ("cw", "ccw"):
                rdmas[(dirn, 0, j)] = make(dirn, 0, j)
                rdmas[(dirn, 0, j)].start()

        for s in range(1, N_DEV - 1):
            c_cw = dev_at(r - 1 - s)
            c_ccw = dev_at(r + 1 + s)
            for j in range(K_SUB):
                lo = j * m_sub
                hi = (j + 1) * m_sub
                xj_cw = cw_x(c_cw, j)
                xj_ccw = ccw_x(c_ccw, j)
                rdmas[("cw", s - 1, j)].wait_recv()
                cw_ref[s - 1, lo:hi, :] = cw_ref[s - 1, lo:hi, :] + xj_cw
                rdmas[("cw", s, j)] = make("cw", s, j)
                rdmas[("cw", s, j)].start()
                rdmas[("ccw", s - 1, j)].wait_recv()
                ccw_ref[s - 1, lo:hi, :] = ccw_ref[s - 1, lo:hi, :] + xj_ccw
                rdmas[("ccw", s, j)] = make("ccw", s, j)
                rdmas[("ccw", s, j)].start()

        for j in range(K_SUB):
            lo = j * m_sub
            hi = (j + 1) * m_sub
            rdmas[("cw", N_DEV - 2, j)].wait_recv()
            out_ref[lo:hi, :] = (
                cw_ref[N_DEV - 2, lo:hi, :].astype(jnp.float32)
                + x_ref[0, lo:hi, pl.ds(my * n_chunk, n_chunk)]
            )
            rdmas[("ccw", N_DEV - 2, j)].wait_recv()
            out_ref[m_half + lo:m_half + hi, :] = (
                ccw_ref[N_DEV - 2, lo:hi, :].astype(jnp.float32)
                + x_ref[0, m_half + lo:m_half + hi, pl.ds(my * n_chunk, n_chunk)]
            )

        for key in rdmas:
            rdmas[key].wait_send()

    return pl.pallas_call(
        body,
        out_shape=jax.ShapeDtypeStruct((m, n_chunk), jnp.float32),
        in_specs=[pl.BlockSpec(memory_space=pltpu.VMEM)],
        out_specs=pl.BlockSpec(memory_space=pltpu.VMEM),
        scratch_shapes=[
            pltpu.VMEM((N_DEV, m_half, n_chunk), jnp.bfloat16),
            pltpu.VMEM((N_DEV, m_half, n_chunk), jnp.bfloat16),
            pltpu.SemaphoreType.DMA((N_DEV - 1, K_SUB)),
            pltpu.SemaphoreType.DMA((N_DEV - 1, K_SUB)),
            pltpu.SemaphoreType.DMA((N_DEV - 1, K_SUB)),
            pltpu.SemaphoreType.DMA((N_DEV - 1, K_SUB)),
        ],
        compiler_params=pltpu.CompilerParams(collective_id=0),
    )(x)


# device time: 54359 ns/iter; 1.0721x vs baseline; 1.0358x over previous
import jax
import jax.numpy as jnp
from jax import lax
from jax.experimental import pallas as pl
from jax.experimental.pallas import tpu as pltpu

N_DEV = 16
N_HOP = N_DEV // 2
K_SUB = 4


def kernel(x):
    _, m, n_total = x.shape
    n_chunk = n_total // N_DEV
    m_sub = m // K_SUB
    half_subs = tuple(range(K_SUB // 2))
    bot_subs = tuple(range(K_SUB // 2, K_SUB))

    def body(
        x_ref, out_ref,
        cw_ref, ccw_ref,
        cw_send, cw_recv, ccw_send, ccw_recv,
    ):
        my = lax.axis_index("i")

        q = lax.rem(my, 4)
        z = my // 4
        r = jnp.where(
            q == 0, z,
            jnp.where(q == 1, 7 - z, jnp.where(q == 2, 8 + z, 15 - z)),
        )

        def dev_at(rho):
            rho = lax.rem(rho + 2 * N_DEV, N_DEV)
            col = rho // 4
            off = lax.rem(rho, 4)
            return jnp.where(
                col == 0, 4 * off,
                jnp.where(
                    col == 1, 4 * (3 - off) + 1,
                    jnp.where(col == 2, 4 * off + 2, 4 * (3 - off) + 3),
                ),
            )

        left = dev_at(r - 1)
        right = dev_at(r + 1)

        barrier_sem = pltpu.get_barrier_semaphore()
        for nbr in (left, right):
            pl.semaphore_signal(
                barrier_sem, inc=1,
                device_id=(nbr,), device_id_type=pl.DeviceIdType.MESH,
            )
        pl.semaphore_wait(barrier_sem, 2)

        def x_rows(c, j):
            return x_ref[
                0, j * m_sub:(j + 1) * m_sub, pl.ds(c * n_chunk, n_chunk)
            ].astype(jnp.bfloat16)

        def make(dirn, s, j):
            buf, send, recv, tgt = {
                "cw": (cw_ref, cw_send, cw_recv, right),
                "ccw": (ccw_ref, ccw_send, ccw_recv, left),
            }[dirn]
            src_slot = N_HOP if s == 0 else (s - 1)
            return pltpu.make_async_remote_copy(
                src_ref=buf.at[src_slot, pl.ds(j * m_sub, m_sub), :],
                dst_ref=buf.at[s, pl.ds(j * m_sub, m_sub), :],
                send_sem=send.at[s, j],
                recv_sem=recv.at[s, j],
                device_id=(tgt,),
                device_id_type=pl.DeviceIdType.MESH,
            )

        anti = dev_at(r + N_HOP)
        rdmas = {}
        for j in half_subs:
            cw_ref[N_HOP, pl.ds(j * m_sub, m_sub), :] = x_rows(anti, j)
            rdmas[("cw", 0, j)] = make("cw", 0, j)
            rdmas[("cw", 0, j)].start()
        for j in bot_subs:
            ccw_ref[N_HOP, pl.ds(j * m_sub, m_sub), :] = x_rows(anti, j)
            rdmas[("ccw", 0, j)] = make("ccw", 0, j)
            rdmas[("ccw", 0, j)].start()

        for s in range(1, N_HOP):
            c_cw = dev_at(r + N_HOP - s)
            c_ccw = dev_at(r - N_HOP + s)
            for j in range(K_SUB):
                lo = j * m_sub
                xj_cw = x_rows(c_cw, j)
                xj_ccw = x_rows(c_ccw, j)
                if s == 1 and j in bot_subs:
                    cw_ref[0, pl.ds(lo, m_sub), :] = xj_cw
                else:
                    rdmas[("cw", s - 1, j)].wait_recv()
                    cw_ref[s - 1, pl.ds(lo, m_sub), :] = (
                        cw_ref[s - 1, pl.ds(lo, m_sub), :] + xj_cw
                    )
                rdmas[("cw", s, j)] = make("cw", s, j)
                rdmas[("cw", s, j)].start()
                if s == 1 and j in half_subs:
                    ccw_ref[0, pl.ds(lo, m_sub), :] = xj_ccw
                else:
                    rdmas[("ccw", s - 1, j)].wait_recv()
                    ccw_ref[s - 1, pl.ds(lo, m_sub), :] = (
                        ccw_ref[s - 1, pl.ds(lo, m_sub), :] + xj_ccw
                    )
                rdmas[("ccw", s, j)] = make("ccw", s, j)
                rdmas[("ccw", s, j)].start()

        for j in range(K_SUB):
            lo = j * m_sub
            rdmas[("cw", N_HOP - 1, j)].wait_recv()
            rdmas[("ccw", N_HOP - 1, j)].wait_recv()
            out_ref[pl.ds(lo, m_sub), :] = (
                x_ref[0, lo:lo + m_sub, pl.ds(my * n_chunk, n_chunk)]
                + cw_ref[N_HOP - 1, pl.ds(lo, m_sub), :].astype(jnp.float32)
                + ccw_ref[N_HOP - 1, pl.ds(lo, m_sub), :].astype(jnp.float32)
            )

        for key in rdmas:
            rdmas[key].wait_send()

    return pl.pallas_call(
        body,
        out_shape=jax.ShapeDtypeStruct((m, n_chunk), jnp.float32),
        in_specs=[pl.BlockSpec(memory_space=pltpu.VMEM)],
        out_specs=pl.BlockSpec(memory_space=pltpu.VMEM),
        scratch_shapes=[
            pltpu.VMEM((N_HOP + 1, m, n_chunk), jnp.bfloat16),
            pltpu.VMEM((N_HOP + 1, m, n_chunk), jnp.bfloat16),
            pltpu.SemaphoreType.DMA((N_HOP, K_SUB)),
            pltpu.SemaphoreType.DMA((N_HOP, K_SUB)),
            pltpu.SemaphoreType.DMA((N_HOP, K_SUB)),
            pltpu.SemaphoreType.DMA((N_HOP, K_SUB)),
        ],
        compiler_params=pltpu.CompilerParams(collective_id=0),
    )(x)
